# Initial kernel scaffold; baseline (speedup 1.0000x reference)
import jax
import jax.numpy as jnp
from jax.experimental import pallas as pl


def kernel(x, edge_index, batch, W1, b1, W2, b2):
    raise NotImplementedError("write your pallas kernel here")


# speedup vs baseline: 29.5809x; 29.5809x over previous
import functools

import jax
import jax.numpy as jnp
from jax import lax
from jax.experimental import pallas as pl
from jax.experimental.pallas import tpu as pltpu
from jax.experimental.pallas import tpu_sc as plsc

N = 10000
E = 320000
D_IN = 128
HIDDEN = 64
EMB = 2
NUM_GRAPHS = 16

NUM_CORES = 2
NUM_SUBCORES = 16
NUM_WORKERS = NUM_CORES * NUM_SUBCORES
EDGES_PER_TILE = E // NUM_WORKERS
CHUNK = 125
NCHUNK = EDGES_PER_TILE // CHUNK
N_PAD = 10240
ROWS_PER_TILE = N_PAD // NUM_SUBCORES
ZCHUNK = 128

f32 = jnp.float32


def _mm_body(x_ref, w_ref, o_ref):
  o_ref[...] = jnp.dot(x_ref[...], w_ref[...], preferred_element_type=f32)


def _matmul(x, w, bm):
  m, k = x.shape
  n = w.shape[1]
  return pl.pallas_call(
      _mm_body,
      grid=(m // bm,),
      in_specs=[
          pl.BlockSpec((bm, k), lambda i: (i, 0)),
          pl.BlockSpec((k, n), lambda i: (0, 0)),
      ],
      out_specs=pl.BlockSpec((bm, n), lambda i: (i, 0)),
      out_shape=jax.ShapeDtypeStruct((m, n), f32),
  )(x, w)


def _sc_fill(buf, nrows, d, val):
  @pl.loop(0, nrows)
  def _(i):
    @pl.loop(0, d, step=16)
    def _(j):
      buf[i, pl.ds(j, 16)] = jnp.full((16,), val, f32)


def _make_sc_scatter(d, with_gather):
  mesh = plsc.VectorSubcoreMesh(core_axis_name="c", subcore_axis_name="s")
  scratch = [
      pltpu.VMEM((NCHUNK, CHUNK), jnp.int32),
      pltpu.VMEM((CHUNK, d), f32),
      pltpu.VMEM((ZCHUNK, d), f32),
      pltpu.VMEM_SHARED((N_PAD, d), f32),
      pltpu.SemaphoreType.DMA,
  ]
  if with_gather:
    scratch = [pltpu.VMEM((NCHUNK, CHUNK), jnp.int32)] + scratch

  def body(*refs):
    if with_gather:
      g_hbm, src_hbm, dst_hbm, out_hbm, srcv, dstv, rows, zbuf, acc, sem = refs
    else:
      dst_hbm, out_hbm, dstv, rows, zbuf, acc, sem = refs
    c = lax.axis_index("c")
    s = lax.axis_index("s")
    w = c * NUM_SUBCORES + s
    pltpu.sync_copy(dst_hbm.at[w], dstv)
    if with_gather:
      pltpu.sync_copy(src_hbm.at[w], srcv)
    _sc_fill(zbuf, ZCHUNK, d, 0.0)

    @pl.loop(0, ROWS_PER_TILE, step=ZCHUNK)
    def _(r):
      pltpu.sync_copy(zbuf, acc.at[pl.ds(s * ROWS_PER_TILE + r, ZCHUNK)])

    plsc.subcore_barrier()
    if not with_gather:
      _sc_fill(rows, CHUNK, d, 1.0)

    @pl.loop(0, NCHUNK)
    def _(j):
      if with_gather:
        pltpu.async_copy(g_hbm.at[srcv.at[j]], rows, sem).wait()
      pltpu.sync_copy(rows, acc.at[dstv.at[j]], add=True)

    plsc.subcore_barrier()
    pltpu.sync_copy(
        acc.at[pl.ds(s * ROWS_PER_TILE, ROWS_PER_TILE)],
        out_hbm.at[c, pl.ds(s * ROWS_PER_TILE, ROWS_PER_TILE)],
    )

  return functools.partial(
      pl.kernel,
      out_type=jax.ShapeDtypeStruct((NUM_CORES, N_PAD, d), f32),
      mesh=mesh,
      scratch_types=scratch,
      compiler_params=pltpu.CompilerParams(use_tc_tiling_on_sc=False),
  )(body)


def _scale_body(h1_ref, dega_ref, degb_ref, o_g1_ref, o_dinv_ref):
  deg = (dega_ref[...] + degb_ref[...])[:N, 0:1] + 1.0
  dinv = lax.rsqrt(deg)
  o_dinv_ref[...] = dinv
  o_g1_ref[...] = h1_ref[...] * dinv


def _layer2_body(s1a_ref, s1b_ref, h1_ref, dinv_ref, b1_ref, w2p_ref,
                 o_g2p_ref, o_h2p_ref):
  dinv = dinv_ref[...]
  s1 = (s1a_ref[...] + s1b_ref[...])[:N]
  out1 = dinv * s1 + (dinv * dinv) * h1_ref[...] + b1_ref[...]
  r = jnp.maximum(out1, 0.0)
  h2p = jnp.dot(r, w2p_ref[...], preferred_element_type=f32)
  o_h2p_ref[...] = h2p
  o_g2p_ref[...] = h2p * dinv


def _pool_body(s2a_ref, s2b_ref, h2p_ref, dinv_ref, batt_ref, b2p_ref, o_ref):
  dinv = dinv_ref[...]
  s2 = (s2a_ref[...] + s2b_ref[...])[:N]
  out2 = dinv * s2 + (dinv * dinv) * h2p_ref[...]
  gids = lax.broadcasted_iota(jnp.int32, (NUM_GRAPHS, N), 0)
  onehot = (batt_ref[...] == gids).astype(f32)
  pooled = jnp.dot(onehot, out2, preferred_element_type=f32)
  counts = jnp.dot(onehot, jnp.ones((N, 1), f32), preferred_element_type=f32)
  o_ref[...] = (pooled + counts * b2p_ref[...]) / jnp.maximum(counts, 1.0)


def _tc_call(body, out_shapes):
  return pl.pallas_call(body, out_shape=out_shapes)


def kernel(x, edge_index, batch, W1, b1, W2, b2):
  x = x.astype(f32)
  ei = edge_index.astype(jnp.int32)
  src = ei[0].reshape(NUM_WORKERS, NCHUNK, CHUNK)
  dst = ei[1].reshape(NUM_WORKERS, NCHUNK, CHUNK)
  batt = batch.astype(jnp.int32).reshape(1, N)
  b1r = b1.reshape(1, HIDDEN)
  w2p = jnp.pad(W2, ((0, 0), (0, 16 - EMB)))
  b2p = jnp.pad(b2, (0, 16 - EMB)).reshape(1, 16)

  h1 = _matmul(x, W1, 1000)
  degp = _make_sc_scatter(16, False)(dst)

  g1, dinv = _tc_call(
      _scale_body,
      (jax.ShapeDtypeStruct((N, HIDDEN), f32), jax.ShapeDtypeStruct((N, 1), f32)),
  )(h1, degp[0], degp[1])

  s1p = _make_sc_scatter(HIDDEN, True)(g1, src, dst)

  g2p, h2p = _tc_call(
      _layer2_body,
      (jax.ShapeDtypeStruct((N, 16), f32), jax.ShapeDtypeStruct((N, 16), f32)),
  )(s1p[0], s1p[1], h1, dinv, b1r, w2p)

  s2p = _make_sc_scatter(16, True)(g2p, src, dst)

  out16 = _tc_call(_pool_body, jax.ShapeDtypeStruct((NUM_GRAPHS, 16), f32))(
      s2p[0], s2p[1], h2p, dinv, batt, b2p)
  return out16[:, :EMB]


# speedup vs baseline: 44.3511x; 1.4993x over previous
import functools

import jax
import jax.numpy as jnp
from jax import lax
from jax.experimental import pallas as pl
from jax.experimental.pallas import tpu as pltpu
from jax.experimental.pallas import tpu_sc as plsc

N = 10000
E = 320000
D_IN = 128
HIDDEN = 64
EMB = 2
NUM_GRAPHS = 16

NUM_CORES = 2
NUM_SUBCORES = 16
NUM_WORKERS = NUM_CORES * NUM_SUBCORES
EDGES_PER_TILE = E // NUM_WORKERS
CHUNK = 125
NCHUNK = EDGES_PER_TILE // CHUNK
N_PAD = 10240
ROWS_PER_TILE = N_PAD // NUM_SUBCORES
ZCHUNK = 128

f32 = jnp.float32


def _sc_fill(buf, nrows, d, val):
  @pl.loop(0, nrows)
  def _(i):
    @pl.loop(0, d, step=16)
    def _(j):
      buf[i, pl.ds(j, 16)] = jnp.full((16,), val, f32)


NBUF = 4


def _make_sc_scatter(d, with_gather):
  mesh = plsc.VectorSubcoreMesh(core_axis_name="c", subcore_axis_name="s")
  scratch = [
      pltpu.VMEM((NCHUNK, CHUNK), jnp.int32),
      pltpu.VMEM((ZCHUNK, d), f32),
      pltpu.VMEM_SHARED((N_PAD, d), f32),
  ]
  if with_gather:
    scratch = ([pltpu.VMEM((NCHUNK, CHUNK), jnp.int32)] + scratch
               + [pltpu.VMEM((CHUNK, d), f32) for _ in range(NBUF)]
               + [pltpu.SemaphoreType.DMA for _ in range(NBUF)])
  else:
    scratch = scratch + [pltpu.VMEM((CHUNK, d), f32)]

  def body(*refs):
    if with_gather:
      g_hbm, src_hbm, dst_hbm, out_hbm, srcv, dstv, zbuf, acc = refs[:8]
      rows = refs[8:8 + NBUF]
      sems = refs[8 + NBUF:8 + 2 * NBUF]
    else:
      dst_hbm, out_hbm, dstv, zbuf, acc, ones = refs
    c = lax.axis_index("c")
    s = lax.axis_index("s")
    w = c * NUM_SUBCORES + s
    pltpu.sync_copy(dst_hbm.at[w], dstv)
    if with_gather:
      pltpu.sync_copy(src_hbm.at[w], srcv)
    _sc_fill(zbuf, ZCHUNK, d, 0.0)

    @pl.loop(0, ROWS_PER_TILE, step=ZCHUNK)
    def _(r):
      pltpu.sync_copy(zbuf, acc.at[pl.ds(s * ROWS_PER_TILE + r, ZCHUNK)])

    plsc.subcore_barrier()

    if with_gather:
      for b in range(NBUF - 1):
        pltpu.async_copy(g_hbm.at[srcv.at[b]], rows[b], sems[b])

      @pl.loop(0, NCHUNK, step=NBUF)
      def _(j):
        for b in range(NBUF):
          nxt = j + b + (NBUF - 1)
          nb = (b + NBUF - 1) % NBUF

          @pl.when(nxt < NCHUNK)
          def _():
            pltpu.async_copy(g_hbm.at[srcv.at[nxt]], rows[nb], sems[nb])

          pltpu.make_async_copy(g_hbm.at[srcv.at[j + b]], rows[b],
                                sems[b]).wait()
          pltpu.sync_copy(rows[b], acc.at[dstv.at[j + b]], add=True)
    else:
      _sc_fill(ones, CHUNK, d, 1.0)

      @pl.loop(0, NCHUNK)
      def _(j):
        pltpu.sync_copy(ones, acc.at[dstv.at[j]], add=True)

    plsc.subcore_barrier()
    pltpu.sync_copy(
        acc.at[pl.ds(s * ROWS_PER_TILE, ROWS_PER_TILE)],
        out_hbm.at[c, pl.ds(s * ROWS_PER_TILE, ROWS_PER_TILE)],
    )

  return functools.partial(
      pl.kernel,
      out_type=jax.ShapeDtypeStruct((NUM_CORES, N_PAD, d), f32),
      mesh=mesh,
      scratch_types=scratch,
      compiler_params=pltpu.CompilerParams(use_tc_tiling_on_sc=False),
  )(body)


def _mm_scale_body(x_ref, w1_ref, dega_ref, degb_ref,
                   o_h1_ref, o_g1_ref, o_dinv_ref):
  h1 = jnp.dot(x_ref[...], w1_ref[...], preferred_element_type=f32)
  deg = dega_ref[...][:, 0:1] + degb_ref[...][:, 0:1] + 1.0
  dinv = lax.rsqrt(deg)
  o_h1_ref[...] = h1
  o_dinv_ref[...] = dinv
  o_g1_ref[...] = h1 * dinv


def _layer2_body(s1a_ref, s1b_ref, h1_ref, dinv_ref, b1_ref, w2p_ref,
                 o_g2p_ref, o_h2p_ref):
  dinv = dinv_ref[...]
  s1 = (s1a_ref[...] + s1b_ref[...])[:N]
  out1 = dinv * s1 + (dinv * dinv) * h1_ref[...] + b1_ref[...]
  r = jnp.maximum(out1, 0.0)
  h2p = jnp.dot(r, w2p_ref[...], preferred_element_type=f32)
  o_h2p_ref[...] = h2p
  o_g2p_ref[...] = h2p * dinv


def _pool_body(s2a_ref, s2b_ref, h2p_ref, dinv_ref, batt_ref, b2p_ref, o_ref):
  dinv = dinv_ref[...]
  s2 = (s2a_ref[...] + s2b_ref[...])[:N]
  out2 = dinv * s2 + (dinv * dinv) * h2p_ref[...]
  gids = lax.broadcasted_iota(jnp.int32, (NUM_GRAPHS, N), 0)
  onehot = (batt_ref[...] == gids).astype(f32)
  pooled = jnp.dot(onehot, out2, preferred_element_type=f32)
  counts = jnp.dot(onehot, jnp.ones((N, 1), f32), preferred_element_type=f32)
  o_ref[...] = (pooled + counts * b2p_ref[...]) / jnp.maximum(counts, 1.0)


def _tc_call(body, out_shapes):
  return pl.pallas_call(body, out_shape=out_shapes)


def kernel(x, edge_index, batch, W1, b1, W2, b2):
  x = x.astype(f32)
  ei = edge_index.astype(jnp.int32)
  src = ei[0].reshape(NUM_WORKERS, NCHUNK, CHUNK)
  dst = ei[1].reshape(NUM_WORKERS, NCHUNK, CHUNK)
  batt = batch.astype(jnp.int32).reshape(1, N)
  b1r = b1.reshape(1, HIDDEN)
  w2p = jnp.pad(W2, ((0, 0), (0, 16 - EMB)))
  b2p = jnp.pad(b2, (0, 16 - EMB)).reshape(1, 16)

  degp = _make_sc_scatter(16, False)(dst)

  bm = 1000
  h1, g1, dinv = pl.pallas_call(
      _mm_scale_body,
      grid=(N // bm,),
      in_specs=[
          pl.BlockSpec((bm, D_IN), lambda i: (i, 0)),
          pl.BlockSpec((D_IN, HIDDEN), lambda i: (0, 0)),
          pl.BlockSpec((bm, 16), lambda i: (i, 0)),
          pl.BlockSpec((bm, 16), lambda i: (i, 0)),
      ],
      out_specs=[
          pl.BlockSpec((bm, HIDDEN), lambda i: (i, 0)),
          pl.BlockSpec((bm, HIDDEN), lambda i: (i, 0)),
          pl.BlockSpec((bm, 1), lambda i: (i, 0)),
      ],
      out_shape=(
          jax.ShapeDtypeStruct((N, HIDDEN), f32),
          jax.ShapeDtypeStruct((N, HIDDEN), f32),
          jax.ShapeDtypeStruct((N, 1), f32),
      ),
  )(x, W1, degp[0], degp[1])

  s1p = _make_sc_scatter(HIDDEN, True)(g1, src, dst)

  g2p, h2p = _tc_call(
      _layer2_body,
      (jax.ShapeDtypeStruct((N, 16), f32), jax.ShapeDtypeStruct((N, 16), f32)),
  )(s1p[0], s1p[1], h1, dinv, b1r, w2p)

  s2p = _make_sc_scatter(16, True)(g2p, src, dst)

  out16 = _tc_call(_pool_body, jax.ShapeDtypeStruct((NUM_GRAPHS, 16), f32))(
      s2p[0], s2p[1], h2p, dinv, batt, b2p)
  return out16[:, :EMB]


# speedup vs baseline: 50.7629x; 1.1446x over previous
import functools

import jax
import jax.numpy as jnp
from jax import lax
from jax.experimental import pallas as pl
from jax.experimental.pallas import tpu as pltpu
from jax.experimental.pallas import tpu_sc as plsc

N = 10000
E = 320000
D_IN = 128
HIDDEN = 64
EMB = 2
NUM_GRAPHS = 16

NUM_CORES = 2
NUM_SUBCORES = 16
NUM_WORKERS = NUM_CORES * NUM_SUBCORES
EDGES_PER_TILE = E // NUM_WORKERS
CHUNK = 125
NCHUNK = EDGES_PER_TILE // CHUNK
N_PAD = 10240
ROWS_PER_TILE = N_PAD // NUM_SUBCORES
ZCHUNK = 128

f32 = jnp.float32


def _sc_fill(buf, nrows, d, val):
  @pl.loop(0, nrows)
  def _(i):
    @pl.loop(0, d, step=16)
    def _(j):
      buf[i, pl.ds(j, 16)] = jnp.full((16,), val, f32)


NBUF = 4
_SC_MESH = plsc.VectorSubcoreMesh(core_axis_name="c", subcore_axis_name="s")
_SC_PARAMS = pltpu.CompilerParams(use_tc_tiling_on_sc=False)


def _zero_acc(acc, zbuf, d, s):
  _sc_fill(zbuf, ZCHUNK, d, 0.0)

  @pl.loop(0, ROWS_PER_TILE, step=ZCHUNK)
  def _(r):
    pltpu.sync_copy(zbuf, acc.at[pl.ds(s * ROWS_PER_TILE + r, ZCHUNK)])


def _copy_out(acc, out_hbm, c, s):
  pltpu.sync_copy(
      acc.at[pl.ds(s * ROWS_PER_TILE, ROWS_PER_TILE)],
      out_hbm.at[c, pl.ds(s * ROWS_PER_TILE, ROWS_PER_TILE)],
  )


def _sc_deg(dst):
  d = 8

  def body(dst_hbm, out_hbm, dstv, zbuf, acc, ones):
    c = lax.axis_index("c")
    s = lax.axis_index("s")
    w = c * NUM_SUBCORES + s
    pltpu.sync_copy(dst_hbm.at[w], dstv)
    _zero_acc(acc, zbuf, d, s)
    plsc.subcore_barrier()
    _sc_fill(ones, CHUNK, d, 1.0)

    @pl.loop(0, NCHUNK)
    def _(j):
      pltpu.sync_copy(ones, acc.at[dstv.at[j]], add=True)

    plsc.subcore_barrier()
    _copy_out(acc, out_hbm, c, s)

  return pl.kernel(
      body,
      out_type=jax.ShapeDtypeStruct((NUM_CORES, N_PAD, d), f32),
      mesh=_SC_MESH,
      scratch_types=[
          pltpu.VMEM((NCHUNK, CHUNK), jnp.int32),
          pltpu.VMEM((ZCHUNK, d), f32),
          pltpu.VMEM_SHARED((N_PAD, d), f32),
          pltpu.VMEM((CHUNK, d), f32),
      ],
      compiler_params=_SC_PARAMS,
  )(dst)


def _sc_dual_scatter(g1, t16, src, dst):
  def body(g_hbm, t_hbm, src_hbm, dst_hbm, s1_hbm, c_hbm,
           srcv, dstv, zb64, zb16, acc64, acc16, *bufs):
    rows64 = bufs[0:NBUF]
    rows16 = bufs[NBUF:2 * NBUF]
    semsA = bufs[2 * NBUF:3 * NBUF]
    semsB = bufs[3 * NBUF:4 * NBUF]
    c = lax.axis_index("c")
    s = lax.axis_index("s")
    w = c * NUM_SUBCORES + s
    pltpu.sync_copy(src_hbm.at[w], srcv)
    pltpu.sync_copy(dst_hbm.at[w], dstv)
    _zero_acc(acc64, zb64, HIDDEN, s)
    _zero_acc(acc16, zb16, 16, s)
    plsc.subcore_barrier()

    for b in range(NBUF - 1):
      pltpu.async_copy(g_hbm.at[srcv.at[b]], rows64[b], semsA[b])
      pltpu.async_copy(t_hbm.at[dstv.at[b]], rows16[b], semsB[b])

    @pl.loop(0, NCHUNK, step=NBUF)
    def _(j):
      for b in range(NBUF):
        nxt = j + b + (NBUF - 1)
        nb = (b + NBUF - 1) % NBUF

        @pl.when(nxt < NCHUNK)
        def _():
          pltpu.async_copy(g_hbm.at[srcv.at[nxt]], rows64[nb], semsA[nb])
          pltpu.async_copy(t_hbm.at[dstv.at[nxt]], rows16[nb], semsB[nb])

        pltpu.make_async_copy(g_hbm.at[srcv.at[j + b]], rows64[b],
                              semsA[b]).wait()
        pltpu.sync_copy(rows64[b], acc64.at[dstv.at[j + b]], add=True)
        pltpu.make_async_copy(t_hbm.at[dstv.at[j + b]], rows16[b],
                              semsB[b]).wait()
        pltpu.sync_copy(rows16[b], acc16.at[srcv.at[j + b]], add=True)

    plsc.subcore_barrier()
    _copy_out(acc64, s1_hbm, c, s)
    _copy_out(acc16, c_hbm, c, s)

  return pl.kernel(
      body,
      out_type=(jax.ShapeDtypeStruct((NUM_CORES, N_PAD, HIDDEN), f32),
                jax.ShapeDtypeStruct((NUM_CORES, N_PAD, 16), f32)),
      mesh=_SC_MESH,
      scratch_types=(
          [pltpu.VMEM((NCHUNK, CHUNK), jnp.int32),
           pltpu.VMEM((NCHUNK, CHUNK), jnp.int32),
           pltpu.VMEM((ZCHUNK, HIDDEN), f32),
           pltpu.VMEM((ZCHUNK, 16), f32),
           pltpu.VMEM_SHARED((N_PAD, HIDDEN), f32),
           pltpu.VMEM_SHARED((N_PAD, 16), f32)]
          + [pltpu.VMEM((CHUNK, HIDDEN), f32) for _ in range(NBUF)]
          + [pltpu.VMEM((CHUNK, 16), f32) for _ in range(NBUF)]
          + [pltpu.SemaphoreType.DMA for _ in range(2 * NBUF)]),
      compiler_params=_SC_PARAMS,
  )(g1, t16, src, dst)


def _tc_a_body(x_ref, w1_ref, dega_ref, degb_ref, batc_ref,
               o_h1_ref, o_g1_ref, o_dinv_ref, o_t_ref):
  h1 = jnp.dot(x_ref[...], w1_ref[...], preferred_element_type=f32)
  deg = dega_ref[...][:, 0:1] + degb_ref[...][:, 0:1] + 1.0
  dinv = lax.rsqrt(deg)
  bm = batc_ref.shape[0]
  gids = lax.broadcasted_iota(jnp.int32, (bm, NUM_GRAPHS), 1)
  onehot = (batc_ref[...] == gids).astype(f32)
  o_h1_ref[...] = h1
  o_dinv_ref[...] = dinv
  o_g1_ref[...] = h1 * dinv
  o_t_ref[...] = onehot * dinv


def _tc_c_body(s1a_ref, s1b_ref, ca_ref, cb_ref, h1_ref, dinv_ref, t_ref,
               batt_ref, b1_ref, w2p_ref, b2p_ref, o_ref):
  dinv = dinv_ref[...]
  s1 = (s1a_ref[...] + s1b_ref[...])[:N]
  out1 = dinv * s1 + (dinv * dinv) * h1_ref[...] + b1_ref[...]
  rp = dinv * jnp.maximum(out1, 0.0)
  cpt = (ca_ref[...] + cb_ref[...])[:N] + t_ref[...]
  m = lax.dot_general(cpt, rp, (((0,), (0,)), ((), ())),
                      preferred_element_type=f32)
  gids = lax.broadcasted_iota(jnp.int32, (NUM_GRAPHS, N), 0)
  onehot = (batt_ref[...] == gids).astype(f32)
  counts = jnp.dot(onehot, jnp.ones((N, 1), f32), preferred_element_type=f32)
  pooled = jnp.dot(m, w2p_ref[...], preferred_element_type=f32)
  o_ref[...] = (pooled + counts * b2p_ref[...]) / jnp.maximum(counts, 1.0)


def kernel(x, edge_index, batch, W1, b1, W2, b2):
  x = x.astype(f32)
  ei = edge_index.astype(jnp.int32)
  src = ei[0].reshape(NUM_WORKERS, NCHUNK, CHUNK)
  dst = ei[1].reshape(NUM_WORKERS, NCHUNK, CHUNK)
  bati = batch.astype(jnp.int32)
  batc = bati.reshape(N, 1)
  batt = bati.reshape(1, N)
  b1r = b1.reshape(1, HIDDEN)
  w2p = jnp.pad(W2, ((0, 0), (0, 16 - EMB)))
  b2p = jnp.pad(b2, (0, 16 - EMB)).reshape(1, 16)

  degp = _sc_deg(dst)

  bm = 1000
  h1, g1, dinv, t16 = pl.pallas_call(
      _tc_a_body,
      grid=(N // bm,),
      in_specs=[
          pl.BlockSpec((bm, D_IN), lambda i: (i, 0)),
          pl.BlockSpec((D_IN, HIDDEN), lambda i: (0, 0)),
          pl.BlockSpec((bm, 8), lambda i: (i, 0)),
          pl.BlockSpec((bm, 8), lambda i: (i, 0)),
          pl.BlockSpec((bm, 1), lambda i: (i, 0)),
      ],
      out_specs=[
          pl.BlockSpec((bm, HIDDEN), lambda i: (i, 0)),
          pl.BlockSpec((bm, HIDDEN), lambda i: (i, 0)),
          pl.BlockSpec((bm, 1), lambda i: (i, 0)),
          pl.BlockSpec((bm, 16), lambda i: (i, 0)),
      ],
      out_shape=(
          jax.ShapeDtypeStruct((N, HIDDEN), f32),
          jax.ShapeDtypeStruct((N, HIDDEN), f32),
          jax.ShapeDtypeStruct((N, 1), f32),
          jax.ShapeDtypeStruct((N, 16), f32),
      ),
  )(x, W1, degp[0], degp[1], batc)

  s1p, cp = _sc_dual_scatter(g1, t16, src, dst)

  out16 = pl.pallas_call(
      _tc_c_body,
      out_shape=jax.ShapeDtypeStruct((NUM_GRAPHS, 16), f32),
  )(s1p[0], s1p[1], cp[0], cp[1], h1, dinv, t16, batt, b1r, w2p, b2p)
  return out16[:, :EMB]
